# baseline (speedup 1.0000x reference)
import functools

import jax
import jax.numpy as jnp
from jax import lax
from jax.experimental import pallas as pl
from jax.experimental.pallas import tpu as pltpu
from jax.experimental.pallas import tpu_sc as plsc

N_NODES = 10000
N_EDGES = 320000
D = 128
N_REL = 200

_NC = 2
_NS = 16
_NW = _NC * _NS
_EPW = N_EDGES // _NW
_C = 80
_NCH = _EPW // _C
_G = _C // 16


def _proj_body(z_ref, whT_ref, wtT_ref, wrT_ref, bh_ref, bt_ref, br_ref,
               er_ref, h_ref, t_ref, r_ref):
    zb = z_ref[...]
    h_ref[...] = jnp.maximum(
        jnp.dot(zb, whT_ref[...], preferred_element_type=jnp.float32)
        + bh_ref[...], 0.0).astype(jnp.bfloat16)
    t_ref[...] = jnp.maximum(
        jnp.dot(zb, wtT_ref[...], preferred_element_type=jnp.float32)
        + bt_ref[...], 0.0).astype(jnp.bfloat16)

    @pl.when(pl.program_id(0) == 0)
    def _():
        r_ref[...] = jnp.maximum(
            jnp.dot(er_ref[...], wrT_ref[...], preferred_element_type=jnp.float32)
            + br_ref[...], 0.0).astype(jnp.bfloat16)


_ZBLK = 1000


def _project(z, er, whT, wtT, wrT, bh, bt, br):
    grid = (N_NODES // _ZBLK,)
    return pl.pallas_call(
        _proj_body,
        grid=grid,
        in_specs=[
            pl.BlockSpec((_ZBLK, D), lambda i: (i, 0)),
            pl.BlockSpec((D, D), lambda i: (0, 0)),
            pl.BlockSpec((D, D), lambda i: (0, 0)),
            pl.BlockSpec((D, D), lambda i: (0, 0)),
            pl.BlockSpec((1, D), lambda i: (0, 0)),
            pl.BlockSpec((1, D), lambda i: (0, 0)),
            pl.BlockSpec((1, D), lambda i: (0, 0)),
            pl.BlockSpec((N_REL, D), lambda i: (0, 0)),
        ],
        out_specs=[
            pl.BlockSpec((_ZBLK, D), lambda i: (i, 0)),
            pl.BlockSpec((_ZBLK, D), lambda i: (i, 0)),
            pl.BlockSpec((N_REL, D), lambda i: (0, 0)),
        ],
        out_shape=[
            jax.ShapeDtypeStruct((N_NODES, D), jnp.bfloat16),
            jax.ShapeDtypeStruct((N_NODES, D), jnp.bfloat16),
            jax.ShapeDtypeStruct((N_REL, D), jnp.bfloat16),
        ],
    )(z, whT, wtT, wrT, bh, bt, br, er)


_mesh = plsc.VectorSubcoreMesh(core_axis_name="c", subcore_axis_name="s")


@functools.partial(
    pl.kernel,
    mesh=_mesh,
    compiler_params=pltpu.CompilerParams(
        needs_layout_passes=False, use_tc_tiling_on_sc=False),
    out_type=[
        jax.ShapeDtypeStruct((_NW, _EPW), jnp.float32),
        jax.ShapeDtypeStruct((_NW, _EPW), jnp.int32),
    ],
    scratch_types=[
        pltpu.VMEM((_EPW,), jnp.int32),
        pltpu.VMEM((_EPW,), jnp.int32),
        pltpu.VMEM((_EPW,), jnp.int32),
        pltpu.VMEM((_C, D // 2), jnp.int32),
        pltpu.VMEM((_C, D // 2), jnp.int32),
        pltpu.VMEM((_C, D // 2), jnp.int32),
        pltpu.VMEM((_C, D // 2), jnp.int32),
        pltpu.VMEM((N_REL, D // 2), jnp.int32),
        pltpu.VMEM((_EPW,), jnp.float32),
        pltpu.VMEM((_EPW,), jnp.int32),
        pltpu.VMEM((272,), jnp.float32),
        pltpu.SemaphoreType.DMA,
        pltpu.SemaphoreType.DMA,
        pltpu.SemaphoreType.DMA,
        pltpu.SemaphoreType.DMA,
    ],
)
def _score(h_hbm, t_hbm, r_hbm, hi_hbm, ti_hbm, ri_hbm, lo_hbm, mk_hbm,
           hi_v, ti_v, ri_v, hr0, hr1, tr0, tr1, rt_v, lo_v, mk_v, tacc,
           sh0, sh1, st0, st1):
    wid = lax.axis_index("s") * _NC + lax.axis_index("c")
    pltpu.sync_copy(r_hbm, rt_v)
    pltpu.sync_copy(hi_hbm.at[wid], hi_v)
    pltpu.sync_copy(ti_hbm.at[wid], ti_v)
    pltpu.sync_copy(ri_hbm.at[wid], ri_v)

    hr = (hr0, hr1)
    tr = (tr0, tr1)
    sh = (sh0, sh1)
    st = (st0, st1)

    def issue(ci, b):
        off = ci * _C
        pltpu.async_copy(h_hbm.at[hi_v.at[pl.ds(off, _C)]], hr[b], sh[b])
        pltpu.async_copy(t_hbm.at[ti_v.at[pl.ds(off, _C)]], tr[b], st[b])

    def wait(b):
        pltpu.make_async_copy(h_hbm.at[hi_v.at[pl.ds(0, _C)]], hr[b], sh[b]).wait()
        pltpu.make_async_copy(t_hbm.at[ti_v.at[pl.ds(0, _C)]], tr[b], st[b]).wait()

    lanes = lax.iota(jnp.int32, 16)

    def compute(ci, b):
        @plsc.parallel_loop(0, _G)
        def group(g):
            e0 = g * 16
            ridx_vec = ri_v[pl.ds(ci * _C + e0, 16)]

            @plsc.parallel_loop(0, 16, unroll=4)
            def edges(l):
                e = e0 + l
                rsplat = jnp.take_along_axis(
                    ridx_vec, jnp.full((16,), l, jnp.int32), axis=0)
                himsk = jnp.int32(-65536)
                acc0 = jnp.zeros((16,), jnp.float32)
                acc1 = jnp.zeros((16,), jnp.float32)
                for q in range(D // 32):
                    hw = hr[b][e, pl.ds(q * 16, 16)]
                    tw = tr[b][e, pl.ds(q * 16, 16)]
                    rw = plsc.load_gather(rt_v, [rsplat, lanes + q * 16])
                    hb = plsc.bitcast(hw, jnp.bfloat16)
                    tb = plsc.bitcast(tw, jnp.bfloat16)
                    rb = plsc.bitcast(rw, jnp.bfloat16)
                    pw = plsc.bitcast((hb * tb) * rb, jnp.int32)
                    acc0 = acc0 + plsc.bitcast(pw << 16, jnp.float32)
                    acc1 = acc1 + plsc.bitcast(pw & himsk, jnp.float32)
                plsc.store_scatter(tacc, [lanes * 17 + l], acc0 + acc1)

            s0 = tacc[pl.ds(0, 16)] + tacc[pl.ds(17, 16)]
            s1 = tacc[pl.ds(34, 16)] + tacc[pl.ds(51, 16)]
            s2 = tacc[pl.ds(68, 16)] + tacc[pl.ds(85, 16)]
            s3 = tacc[pl.ds(102, 16)] + tacc[pl.ds(119, 16)]
            s4 = tacc[pl.ds(136, 16)] + tacc[pl.ds(153, 16)]
            s5 = tacc[pl.ds(170, 16)] + tacc[pl.ds(187, 16)]
            s6 = tacc[pl.ds(204, 16)] + tacc[pl.ds(221, 16)]
            s7 = tacc[pl.ds(238, 16)] + tacc[pl.ds(255, 16)]
            svec = ((s0 + s1) + (s2 + s3)) + ((s4 + s5) + (s6 + s7))
            logit = 1.0 / (1.0 + jnp.exp(-svec))
            eo = ci * _C + e0
            lo_v[pl.ds(eo, 16)] = logit
            mk_v[pl.ds(eo, 16)] = (logit >= 0.5).astype(jnp.int32)

    issue(0, 0)

    def body2(k, carry):
        ci0 = 2 * k
        issue(ci0 + 1, 1)
        wait(0)
        compute(ci0, 0)
        issue(ci0 + 2, 0)
        wait(1)
        compute(ci0 + 1, 1)
        return carry

    lax.fori_loop(0, (_NCH - 1) // 2, body2, 0)
    wait(0)
    compute(_NCH - 1, 0)

    pltpu.sync_copy(lo_v, lo_hbm.at[wid])
    pltpu.sync_copy(mk_v, mk_hbm.at[wid])


def _pack(a):
    n = a.shape[0]
    return jax.lax.bitcast_convert_type(a.reshape(n, D // 2, 2), jnp.int32)


def kernel(z, emb_rel, W_head, b_head, W_tail, b_tail, W_rel, b_rel,
           edge_index, rel_type):
    h, t, r = _project(
        z, emb_rel, W_head.T, W_tail.T, W_rel.T,
        b_head.reshape(1, D), b_tail.reshape(1, D), b_rel.reshape(1, D))
    logit, mask = _score(
        _pack(h), _pack(t), _pack(r),
        edge_index[0].reshape(_NW, _EPW),
        edge_index[1].reshape(_NW, _EPW),
        rel_type.reshape(_NW, _EPW))
    return (logit.reshape(N_EDGES), mask.reshape(N_EDGES))


# speedup vs baseline: 1.0113x; 1.0113x over previous
import functools

import jax
import jax.numpy as jnp
from jax import lax
from jax.experimental import pallas as pl
from jax.experimental.pallas import tpu as pltpu
from jax.experimental.pallas import tpu_sc as plsc

N_NODES = 10000
N_EDGES = 320000
D = 128
N_REL = 200

_NC = 2
_NS = 16
_NW = _NC * _NS
_EPW = N_EDGES // _NW
_C = 80
_NCH = _EPW // _C
_G = _C // 16


def _proj_body(z_ref, whT_ref, wtT_ref, wrT_ref, bh_ref, bt_ref, br_ref,
               er_ref, h_ref, t_ref, r_ref):
    zb = z_ref[...]
    h_ref[...] = jnp.maximum(
        jnp.dot(zb, whT_ref[...], preferred_element_type=jnp.float32)
        + bh_ref[...], 0.0).astype(jnp.bfloat16)
    t_ref[...] = jnp.maximum(
        jnp.dot(zb, wtT_ref[...], preferred_element_type=jnp.float32)
        + bt_ref[...], 0.0).astype(jnp.bfloat16)

    @pl.when(pl.program_id(0) == 0)
    def _():
        r_ref[...] = jnp.maximum(
            jnp.dot(er_ref[...], wrT_ref[...], preferred_element_type=jnp.float32)
            + br_ref[...], 0.0).astype(jnp.bfloat16)


_ZBLK = 1000


def _project(z, er, whT, wtT, wrT, bh, bt, br):
    grid = (N_NODES // _ZBLK,)
    return pl.pallas_call(
        _proj_body,
        grid=grid,
        in_specs=[
            pl.BlockSpec((_ZBLK, D), lambda i: (i, 0)),
            pl.BlockSpec((D, D), lambda i: (0, 0)),
            pl.BlockSpec((D, D), lambda i: (0, 0)),
            pl.BlockSpec((D, D), lambda i: (0, 0)),
            pl.BlockSpec((1, D), lambda i: (0, 0)),
            pl.BlockSpec((1, D), lambda i: (0, 0)),
            pl.BlockSpec((1, D), lambda i: (0, 0)),
            pl.BlockSpec((N_REL, D), lambda i: (0, 0)),
        ],
        out_specs=[
            pl.BlockSpec((_ZBLK, D), lambda i: (i, 0)),
            pl.BlockSpec((_ZBLK, D), lambda i: (i, 0)),
            pl.BlockSpec((N_REL, D), lambda i: (0, 0)),
        ],
        out_shape=[
            jax.ShapeDtypeStruct((N_NODES, D), jnp.bfloat16),
            jax.ShapeDtypeStruct((N_NODES, D), jnp.bfloat16),
            jax.ShapeDtypeStruct((N_REL, D), jnp.bfloat16),
        ],
    )(z, whT, wtT, wrT, bh, bt, br, er)


_mesh = plsc.VectorSubcoreMesh(core_axis_name="c", subcore_axis_name="s")


@functools.partial(
    pl.kernel,
    mesh=_mesh,
    compiler_params=pltpu.CompilerParams(
        needs_layout_passes=False, use_tc_tiling_on_sc=False),
    out_type=[
        jax.ShapeDtypeStruct((_NW, _EPW), jnp.float32),
        jax.ShapeDtypeStruct((_NW, _EPW), jnp.int32),
    ],
    scratch_types=[
        pltpu.VMEM((_EPW,), jnp.int32),
        pltpu.VMEM((_EPW,), jnp.int32),
        pltpu.VMEM((_EPW,), jnp.int32),
        pltpu.VMEM((_C, D // 2), jnp.int32),
        pltpu.VMEM((_C, D // 2), jnp.int32),
        pltpu.VMEM((_C, D // 2), jnp.int32),
        pltpu.VMEM((_C, D // 2), jnp.int32),
        pltpu.VMEM((N_REL, D // 2), jnp.int32),
        pltpu.VMEM((_EPW,), jnp.float32),
        pltpu.VMEM((_EPW,), jnp.int32),
        pltpu.VMEM((272,), jnp.float32),
        pltpu.SemaphoreType.DMA,
        pltpu.SemaphoreType.DMA,
        pltpu.SemaphoreType.DMA,
        pltpu.SemaphoreType.DMA,
    ],
)
def _score(h_hbm, t_hbm, r_hbm, hi_hbm, ti_hbm, ri_hbm, lo_hbm, mk_hbm,
           hi_v, ti_v, ri_v, hr0, hr1, tr0, tr1, rt_v, lo_v, mk_v, tacc,
           sh0, sh1, st0, st1):
    wid = lax.axis_index("s") * _NC + lax.axis_index("c")
    pltpu.sync_copy(r_hbm, rt_v)
    pltpu.sync_copy(hi_hbm.at[wid], hi_v)
    pltpu.sync_copy(ti_hbm.at[wid], ti_v)
    pltpu.sync_copy(ri_hbm.at[wid], ri_v)

    hr = (hr0, hr1)
    tr = (tr0, tr1)
    sh = (sh0, sh1)
    st = (st0, st1)

    def issue(ci, b):
        off = ci * _C
        pltpu.async_copy(h_hbm.at[hi_v.at[pl.ds(off, _C)]], hr[b], sh[b])
        pltpu.async_copy(t_hbm.at[ti_v.at[pl.ds(off, _C)]], tr[b], st[b])

    def wait(b):
        pltpu.make_async_copy(h_hbm.at[hi_v.at[pl.ds(0, _C)]], hr[b], sh[b]).wait()
        pltpu.make_async_copy(t_hbm.at[ti_v.at[pl.ds(0, _C)]], tr[b], st[b]).wait()

    lanes = lax.iota(jnp.int32, 16)

    def compute(ci, b):
        @plsc.parallel_loop(0, _G)
        def group(g):
            e0 = g * 16
            ridx_vec = ri_v[pl.ds(ci * _C + e0, 16)]

            @plsc.parallel_loop(0, 16, unroll=1)
            def edges(l):
                e = e0 + l
                rsplat = jnp.take_along_axis(
                    ridx_vec, jnp.full((16,), l, jnp.int32), axis=0)
                himsk = jnp.int32(-65536)
                acc0 = jnp.zeros((16,), jnp.float32)
                acc1 = jnp.zeros((16,), jnp.float32)
                for q in range(D // 32):
                    hw = hr[b][e, pl.ds(q * 16, 16)]
                    tw = tr[b][e, pl.ds(q * 16, 16)]
                    rw = plsc.load_gather(rt_v, [rsplat, lanes + q * 16])
                    hb = plsc.bitcast(hw, jnp.bfloat16)
                    tb = plsc.bitcast(tw, jnp.bfloat16)
                    rb = plsc.bitcast(rw, jnp.bfloat16)
                    pw = plsc.bitcast((hb * tb) * rb, jnp.int32)
                    acc0 = acc0 + plsc.bitcast(pw << 16, jnp.float32)
                    acc1 = acc1 + plsc.bitcast(pw & himsk, jnp.float32)
                plsc.store_scatter(tacc, [lanes * 17 + l], acc0 + acc1)

            s0 = tacc[pl.ds(0, 16)] + tacc[pl.ds(17, 16)]
            s1 = tacc[pl.ds(34, 16)] + tacc[pl.ds(51, 16)]
            s2 = tacc[pl.ds(68, 16)] + tacc[pl.ds(85, 16)]
            s3 = tacc[pl.ds(102, 16)] + tacc[pl.ds(119, 16)]
            s4 = tacc[pl.ds(136, 16)] + tacc[pl.ds(153, 16)]
            s5 = tacc[pl.ds(170, 16)] + tacc[pl.ds(187, 16)]
            s6 = tacc[pl.ds(204, 16)] + tacc[pl.ds(221, 16)]
            s7 = tacc[pl.ds(238, 16)] + tacc[pl.ds(255, 16)]
            svec = ((s0 + s1) + (s2 + s3)) + ((s4 + s5) + (s6 + s7))
            logit = 1.0 / (1.0 + jnp.exp(-svec))
            eo = ci * _C + e0
            lo_v[pl.ds(eo, 16)] = logit
            mk_v[pl.ds(eo, 16)] = (logit >= 0.5).astype(jnp.int32)

    issue(0, 0)

    def body2(k, carry):
        ci0 = 2 * k
        issue(ci0 + 1, 1)
        wait(0)
        compute(ci0, 0)
        issue(ci0 + 2, 0)
        wait(1)
        compute(ci0 + 1, 1)
        return carry

    lax.fori_loop(0, (_NCH - 1) // 2, body2, 0)
    wait(0)
    compute(_NCH - 1, 0)

    pltpu.sync_copy(lo_v, lo_hbm.at[wid])
    pltpu.sync_copy(mk_v, mk_hbm.at[wid])


def _pack(a):
    n = a.shape[0]
    return jax.lax.bitcast_convert_type(a.reshape(n, D // 2, 2), jnp.int32)


def kernel(z, emb_rel, W_head, b_head, W_tail, b_tail, W_rel, b_rel,
           edge_index, rel_type):
    h, t, r = _project(
        z, emb_rel, W_head.T, W_tail.T, W_rel.T,
        b_head.reshape(1, D), b_tail.reshape(1, D), b_rel.reshape(1, D))
    logit, mask = _score(
        _pack(h), _pack(t), _pack(r),
        edge_index[0].reshape(_NW, _EPW),
        edge_index[1].reshape(_NW, _EPW),
        rel_type.reshape(_NW, _EPW))
    return (logit.reshape(N_EDGES), mask.reshape(N_EDGES))


# speedup vs baseline: 1.3292x; 1.3144x over previous
import functools

import jax
import jax.numpy as jnp
from jax import lax
from jax.experimental import pallas as pl
from jax.experimental.pallas import tpu as pltpu
from jax.experimental.pallas import tpu_sc as plsc

N_NODES = 10000
N_EDGES = 320000
D = 128
N_REL = 200

_NC = 2
_NS = 16
_NW = _NC * _NS
_EPW = N_EDGES // _NW
_C = 80
_NCH = _EPW // _C
_G = _C // 16


def _rne_bf16_bits(x):
    b = jax.lax.bitcast_convert_type(x, jnp.int32)
    return jax.lax.shift_right_logical(
        b + jnp.int32(0x7FFF) +
        (jax.lax.shift_right_logical(b, 16) & jnp.int32(1)), 16)


def _pack_rows(x):
    lo = _rne_bf16_bits(x[:, : D // 2])
    hi = _rne_bf16_bits(x[:, D // 2:])
    return lo | (hi << 16)


def _proj_body(z_ref, whT_ref, wtT_ref, wrT_ref, bh_ref, bt_ref, br_ref,
               er_ref, h_ref, t_ref, r_ref):
    zb = z_ref[...]
    h_ref[...] = _pack_rows(jnp.maximum(
        jnp.dot(zb, whT_ref[...], preferred_element_type=jnp.float32)
        + bh_ref[...], 0.0))
    t_ref[...] = _pack_rows(jnp.maximum(
        jnp.dot(zb, wtT_ref[...], preferred_element_type=jnp.float32)
        + bt_ref[...], 0.0))

    @pl.when(pl.program_id(0) == 0)
    def _():
        r_ref[...] = _pack_rows(jnp.maximum(
            jnp.dot(er_ref[...], wrT_ref[...], preferred_element_type=jnp.float32)
            + br_ref[...], 0.0))


_ZBLK = 1000


def _project(z, er, whT, wtT, wrT, bh, bt, br):
    grid = (N_NODES // _ZBLK,)
    return pl.pallas_call(
        _proj_body,
        grid=grid,
        in_specs=[
            pl.BlockSpec((_ZBLK, D), lambda i: (i, 0)),
            pl.BlockSpec((D, D), lambda i: (0, 0)),
            pl.BlockSpec((D, D), lambda i: (0, 0)),
            pl.BlockSpec((D, D), lambda i: (0, 0)),
            pl.BlockSpec((1, D), lambda i: (0, 0)),
            pl.BlockSpec((1, D), lambda i: (0, 0)),
            pl.BlockSpec((1, D), lambda i: (0, 0)),
            pl.BlockSpec((N_REL, D), lambda i: (0, 0)),
        ],
        out_specs=[
            pl.BlockSpec((_ZBLK, D // 2), lambda i: (i, 0)),
            pl.BlockSpec((_ZBLK, D // 2), lambda i: (i, 0)),
            pl.BlockSpec((N_REL, D // 2), lambda i: (0, 0)),
        ],
        out_shape=[
            jax.ShapeDtypeStruct((N_NODES, D // 2), jnp.int32),
            jax.ShapeDtypeStruct((N_NODES, D // 2), jnp.int32),
            jax.ShapeDtypeStruct((N_REL, D // 2), jnp.int32),
        ],
    )(z, whT, wtT, wrT, bh, bt, br, er)


_mesh = plsc.VectorSubcoreMesh(core_axis_name="c", subcore_axis_name="s")


@functools.partial(
    pl.kernel,
    mesh=_mesh,
    compiler_params=pltpu.CompilerParams(
        needs_layout_passes=False, use_tc_tiling_on_sc=False),
    out_type=[
        jax.ShapeDtypeStruct((_NW, _EPW), jnp.float32),
        jax.ShapeDtypeStruct((_NW, _EPW), jnp.int32),
    ],
    scratch_types=[
        pltpu.VMEM((_EPW,), jnp.int32),
        pltpu.VMEM((_EPW,), jnp.int32),
        pltpu.VMEM((_EPW,), jnp.int32),
        pltpu.VMEM((_C, D // 2), jnp.int32),
        pltpu.VMEM((_C, D // 2), jnp.int32),
        pltpu.VMEM((_C, D // 2), jnp.int32),
        pltpu.VMEM((_C, D // 2), jnp.int32),
        pltpu.VMEM((N_REL, D // 2), jnp.int32),
        pltpu.VMEM((_EPW,), jnp.float32),
        pltpu.VMEM((_EPW,), jnp.int32),
        pltpu.VMEM((272,), jnp.float32),
        pltpu.SemaphoreType.DMA,
        pltpu.SemaphoreType.DMA,
        pltpu.SemaphoreType.DMA,
        pltpu.SemaphoreType.DMA,
    ],
)
def _score(h_hbm, t_hbm, r_hbm, hi_hbm, ti_hbm, ri_hbm, lo_hbm, mk_hbm,
           hi_v, ti_v, ri_v, hr0, hr1, tr0, tr1, rt_v, lo_v, mk_v, tacc,
           sh0, sh1, st0, st1):
    wid = lax.axis_index("s") * _NC + lax.axis_index("c")
    pltpu.sync_copy(r_hbm, rt_v)
    pltpu.sync_copy(hi_hbm.at[wid], hi_v)
    pltpu.sync_copy(ti_hbm.at[wid], ti_v)
    pltpu.sync_copy(ri_hbm.at[wid], ri_v)

    hr = (hr0, hr1)
    tr = (tr0, tr1)
    sh = (sh0, sh1)
    st = (st0, st1)

    def issue(ci, b):
        off = ci * _C
        pltpu.async_copy(h_hbm.at[hi_v.at[pl.ds(off, _C)]], hr[b], sh[b])
        pltpu.async_copy(t_hbm.at[ti_v.at[pl.ds(off, _C)]], tr[b], st[b])

    def wait(b):
        pltpu.make_async_copy(h_hbm.at[hi_v.at[pl.ds(0, _C)]], hr[b], sh[b]).wait()
        pltpu.make_async_copy(t_hbm.at[ti_v.at[pl.ds(0, _C)]], tr[b], st[b]).wait()

    lanes = lax.iota(jnp.int32, 16)

    def compute(ci, b):
        @plsc.parallel_loop(0, _G)
        def group(g):
            e0 = g * 16
            ridx_vec = ri_v[pl.ds(ci * _C + e0, 16)]

            @plsc.parallel_loop(0, 16, unroll=1)
            def edges(l):
                e = e0 + l
                rsplat = jnp.take_along_axis(
                    ridx_vec, jnp.full((16,), l, jnp.int32), axis=0)
                himsk = jnp.int32(-65536)
                acc0 = jnp.zeros((16,), jnp.float32)
                acc1 = jnp.zeros((16,), jnp.float32)
                for q in range(D // 32):
                    hw = hr[b][e, pl.ds(q * 16, 16)]
                    tw = tr[b][e, pl.ds(q * 16, 16)]
                    rw = plsc.load_gather(rt_v, [rsplat, lanes + q * 16])
                    hb = plsc.bitcast(hw, jnp.bfloat16)
                    tb = plsc.bitcast(tw, jnp.bfloat16)
                    rb = plsc.bitcast(rw, jnp.bfloat16)
                    pw = plsc.bitcast((hb * tb) * rb, jnp.int32)
                    acc0 = acc0 + plsc.bitcast(pw << 16, jnp.float32)
                    acc1 = acc1 + plsc.bitcast(pw & himsk, jnp.float32)
                plsc.store_scatter(tacc, [lanes * 17 + l], acc0 + acc1)

            s0 = tacc[pl.ds(0, 16)] + tacc[pl.ds(17, 16)]
            s1 = tacc[pl.ds(34, 16)] + tacc[pl.ds(51, 16)]
            s2 = tacc[pl.ds(68, 16)] + tacc[pl.ds(85, 16)]
            s3 = tacc[pl.ds(102, 16)] + tacc[pl.ds(119, 16)]
            s4 = tacc[pl.ds(136, 16)] + tacc[pl.ds(153, 16)]
            s5 = tacc[pl.ds(170, 16)] + tacc[pl.ds(187, 16)]
            s6 = tacc[pl.ds(204, 16)] + tacc[pl.ds(221, 16)]
            s7 = tacc[pl.ds(238, 16)] + tacc[pl.ds(255, 16)]
            svec = ((s0 + s1) + (s2 + s3)) + ((s4 + s5) + (s6 + s7))
            logit = 1.0 / (1.0 + jnp.exp(-svec))
            eo = ci * _C + e0
            lo_v[pl.ds(eo, 16)] = logit
            mk_v[pl.ds(eo, 16)] = (logit >= 0.5).astype(jnp.int32)

    issue(0, 0)

    def body2(k, carry):
        ci0 = 2 * k
        issue(ci0 + 1, 1)
        wait(0)
        compute(ci0, 0)
        issue(ci0 + 2, 0)
        wait(1)
        compute(ci0 + 1, 1)
        return carry

    lax.fori_loop(0, (_NCH - 1) // 2, body2, 0)
    wait(0)
    compute(_NCH - 1, 0)

    pltpu.sync_copy(lo_v, lo_hbm.at[wid])
    pltpu.sync_copy(mk_v, mk_hbm.at[wid])


def kernel(z, emb_rel, W_head, b_head, W_tail, b_tail, W_rel, b_rel,
           edge_index, rel_type):
    h, t, r = _project(
        z, emb_rel, W_head.T, W_tail.T, W_rel.T,
        b_head.reshape(1, D), b_tail.reshape(1, D), b_rel.reshape(1, D))
    logit, mask = _score(
        h, t, r,
        edge_index[0].reshape(_NW, _EPW),
        edge_index[1].reshape(_NW, _EPW),
        rel_type.reshape(_NW, _EPW))
    return (logit.reshape(N_EDGES), mask.reshape(N_EDGES))
